# Initial kernel scaffold; baseline (speedup 1.0000x reference)
import jax
import jax.numpy as jnp
from jax.experimental import pallas as pl


def kernel(x, norm1_w, Wq, bq, Wk, bk, Wv, bv, Wo, bo, norm2_w, Wr, br, W1, b1, W2, b2):
    raise NotImplementedError("write your pallas kernel here")


# speedup vs baseline: 1.8129x; 1.8129x over previous
import functools

import jax
import jax.numpy as jnp
from jax.experimental import pallas as pl
from jax.experimental.pallas import tpu as pltpu

D = 768
H = 12
DH = 64
E = 8
FF = 3072
S = 2048
TM = 256
NT = S // TM

_INTERP = False


def _rms(x, w):
    return x * jax.lax.rsqrt(jnp.mean(x * x, axis=-1, keepdims=True) + 1e-6) * w


def _qkv_body(x_ref, w1_ref, wq_ref, wk_ref, wv_ref, bq_ref, bk_ref, bv_ref,
              cos_ref, sin_ref, q_ref, k_ref, v_ref):
    x = x_ref[...]
    h = _rms(x, w1_ref[...])
    q = jnp.dot(h, wq_ref[...], preferred_element_type=jnp.float32) + bq_ref[...]
    k = jnp.dot(h, wk_ref[...], preferred_element_type=jnp.float32) + bk_ref[...]
    v = jnp.dot(h, wv_ref[...], preferred_element_type=jnp.float32) + bv_ref[...]
    cos = cos_ref[...]
    sin = sin_ref[...]
    lane = jax.lax.broadcasted_iota(jnp.int32, (TM, D), 1)
    lo = (lane % DH) < (DH // 2)
    z = jnp.zeros((TM, DH // 2), jnp.float32)

    def rot(t):
        tl = jnp.concatenate([t[:, DH // 2:], z], axis=1)
        tr = jnp.concatenate([z, t[:, :D - DH // 2]], axis=1)
        return jnp.where(lo, -tl, tr)

    q_ref[...] = q * cos + rot(q) * sin
    k_ref[...] = k * cos + rot(k) * sin
    v_ref[...] = v


def _qkv(x2, norm1_w, Wq, Wk, Wv, bq, bk, bv, cosf, sinf):
    full = pl.BlockSpec((D, D), lambda i: (0, 0))
    row = pl.BlockSpec((1, D), lambda i: (0, 0))
    tile = pl.BlockSpec((TM, D), lambda i: (i, 0))
    return pl.pallas_call(
        _qkv_body,
        grid=(NT,),
        in_specs=[tile, row, full, full, full, row, row, row, tile, tile],
        out_specs=[tile, tile, tile],
        out_shape=[jax.ShapeDtypeStruct((S, D), jnp.float32)] * 3,
        interpret=_INTERP,
    )(x2, norm1_w, Wq, Wk, Wv, bq, bk, bv, cosf, sinf)


def _attn_body(q_ref, k_ref, v_ref, o_ref):
    i = pl.program_id(1)
    q = q_ref[0]
    k = k_ref[0]
    v = v_ref[0]
    s = jax.lax.dot_general(q, k, (((1,), (1,)), ((), ())),
                            preferred_element_type=jnp.float32) * 0.125
    rowi = i * TM + jax.lax.broadcasted_iota(jnp.int32, (TM, S), 0)
    coli = jax.lax.broadcasted_iota(jnp.int32, (TM, S), 1)
    s = jnp.where(coli <= rowi, s, -1e9)
    m = jnp.max(s, axis=1, keepdims=True)
    p = jnp.exp(s - m)
    l = jnp.sum(p, axis=1, keepdims=True)
    o = jnp.dot(p, v, preferred_element_type=jnp.float32)
    o_ref[0] = o / l


def _attn(q3, k3, v3):
    return pl.pallas_call(
        _attn_body,
        grid=(H, NT),
        in_specs=[
            pl.BlockSpec((1, TM, DH), lambda h, i: (h, i, 0)),
            pl.BlockSpec((1, S, DH), lambda h, i: (h, 0, 0)),
            pl.BlockSpec((1, S, DH), lambda h, i: (h, 0, 0)),
        ],
        out_specs=pl.BlockSpec((1, TM, DH), lambda h, i: (h, i, 0)),
        out_shape=jax.ShapeDtypeStruct((H, S, DH), jnp.float32),
        interpret=_INTERP,
    )(q3, k3, v3)


def _post_body(o_ref, x_ref, wo_ref, bo_ref, w2n_ref, wr_ref, br_ref,
               h1_ref, hn_ref, rl_ref, ti_ref, tw_ref, cb_ref):
    att = jnp.dot(o_ref[...], wo_ref[...],
                  preferred_element_type=jnp.float32) + bo_ref[...]
    h1 = x_ref[...] + att
    h1_ref[...] = h1
    hn = _rms(h1, w2n_ref[...])
    hn_ref[...] = hn
    rl = jnp.dot(hn, wr_ref[...], preferred_element_type=jnp.float32) + br_ref[...]
    rl_ref[...] = rl
    ii = jax.lax.broadcasted_iota(jnp.int32, (TM, E), 1)
    m1 = jnp.max(rl, axis=1, keepdims=True)
    i1 = jnp.min(jnp.where(rl == m1, ii, E), axis=1, keepdims=True)
    ml = jnp.where(ii == i1, -1e30, rl)
    m2 = jnp.max(ml, axis=1, keepdims=True)
    i2 = jnp.min(jnp.where(ml == m2, ii, E), axis=1, keepdims=True)
    e2 = jnp.exp(m2 - m1)
    w1 = 1.0 / (1.0 + e2)
    w2 = e2 / (1.0 + e2)
    ti_ref[...] = jnp.concatenate([i1, i2], axis=1)
    tw_ref[...] = jnp.concatenate([w1, w2], axis=1)
    cb_ref[...] = (jnp.where(ii == i1, w1, 0.0) + jnp.where(ii == i2, w2, 0.0))


def _post(o768, x2, Wo, bo, norm2_w, Wr, br):
    tile = pl.BlockSpec((TM, D), lambda i: (i, 0))
    return pl.pallas_call(
        _post_body,
        grid=(NT,),
        in_specs=[
            tile, tile,
            pl.BlockSpec((D, D), lambda i: (0, 0)),
            pl.BlockSpec((1, D), lambda i: (0, 0)),
            pl.BlockSpec((1, D), lambda i: (0, 0)),
            pl.BlockSpec((D, E), lambda i: (0, 0)),
            pl.BlockSpec((1, E), lambda i: (0, 0)),
        ],
        out_specs=[
            tile, tile,
            pl.BlockSpec((TM, E), lambda i: (i, 0)),
            pl.BlockSpec((TM, 2), lambda i: (i, 0)),
            pl.BlockSpec((TM, 2), lambda i: (i, 0)),
            pl.BlockSpec((TM, E), lambda i: (i, 0)),
        ],
        out_shape=[
            jax.ShapeDtypeStruct((S, D), jnp.float32),
            jax.ShapeDtypeStruct((S, D), jnp.float32),
            jax.ShapeDtypeStruct((S, E), jnp.float32),
            jax.ShapeDtypeStruct((S, 2), jnp.int32),
            jax.ShapeDtypeStruct((S, 2), jnp.float32),
            jax.ShapeDtypeStruct((S, E), jnp.float32),
        ],
        interpret=_INTERP,
    )(o768, x2, Wo, bo, norm2_w, Wr, br)


def _moe_dense_body(hn_ref, w1_ref, b1_ref, w2_ref, b2_ref, c_ref, out_ref,
                    acc_ref):
    e = pl.program_id(0)
    i = pl.program_id(1)

    g = jnp.dot(hn_ref[...], w1_ref[0],
                preferred_element_type=jnp.float32) + b1_ref[0]
    g = 0.5 * g * (1.0 + jax.lax.erf(g * (2.0 ** -0.5)))
    oe = jnp.dot(g, w2_ref[0], preferred_element_type=jnp.float32) + b2_ref[0]
    lane = jax.lax.broadcasted_iota(jnp.int32, (TM, E), 1)
    c = jnp.sum(jnp.where(lane == e, c_ref[...], 0.0), axis=1, keepdims=True)
    part = c * oe

    @pl.when(e == 0)
    def _():
        acc_ref[pl.ds(i * TM, TM), :] = part

    @pl.when(jnp.logical_and(e > 0, e < E - 1))
    def _():
        acc_ref[pl.ds(i * TM, TM), :] += part

    @pl.when(e == E - 1)
    def _():
        out_ref[...] = acc_ref[pl.ds(i * TM, TM), :] + part


def _moe_dense(hn, W1, b1, W2, b2, comb):
    return pl.pallas_call(
        _moe_dense_body,
        grid=(E, NT),
        in_specs=[
            pl.BlockSpec((TM, D), lambda e, i: (i, 0)),
            pl.BlockSpec((1, D, FF), lambda e, i: (e, 0, 0)),
            pl.BlockSpec((1, 1, FF), lambda e, i: (e, 0, 0)),
            pl.BlockSpec((1, FF, D), lambda e, i: (e, 0, 0)),
            pl.BlockSpec((1, 1, D), lambda e, i: (e, 0, 0)),
            pl.BlockSpec((TM, E), lambda e, i: (i, 0)),
        ],
        out_specs=pl.BlockSpec((TM, D), lambda e, i: (i, 0)),
        out_shape=jax.ShapeDtypeStruct((S, D), jnp.float32),
        scratch_shapes=[pltpu.VMEM((S, D), jnp.float32)],
        compiler_params=pltpu.CompilerParams(
            dimension_semantics=("arbitrary", "arbitrary")),
        interpret=_INTERP,
    )(hn, W1.reshape(E, D, FF), b1.reshape(E, 1, FF),
      W2.reshape(E, FF, D), b2.reshape(E, 1, D), comb)


def kernel(x, norm1_w, Wq, bq, Wk, bk, Wv, bv, Wo, bo, norm2_w, Wr, br,
           W1, b1, W2, b2):
    x2 = x.reshape(S, D)
    inv_freq = 1.0 / (10000.0 ** (jnp.arange(0, DH, 2, dtype=jnp.float32) / DH))
    freqs = jnp.arange(S, dtype=jnp.float32)[:, None] * inv_freq[None, :]
    emb = jnp.concatenate([freqs, freqs], axis=-1)
    cosf = jnp.tile(jnp.cos(emb), (1, H))
    sinf = jnp.tile(jnp.sin(emb), (1, H))

    q, k, v = _qkv(x2, norm1_w.reshape(1, D), Wq, Wk, Wv,
                   bq.reshape(1, D), bk.reshape(1, D), bv.reshape(1, D),
                   cosf, sinf)
    q3 = q.reshape(S, H, DH).transpose(1, 0, 2)
    k3 = k.reshape(S, H, DH).transpose(1, 0, 2)
    v3 = v.reshape(S, H, DH).transpose(1, 0, 2)
    o3 = _attn(q3, k3, v3)
    o768 = o3.transpose(1, 0, 2).reshape(S, D)

    h1, hn, rl, ti, tw, comb = _post(o768, x2, Wo, bo.reshape(1, D),
                                     norm2_w.reshape(1, D), Wr,
                                     br.reshape(1, E))

    moe = _moe_dense(hn, W1, b1, W2, b2, comb)
    out = (h1 + moe).reshape(1, S, D)
    return out, rl
